# baseline (speedup 1.0000x reference)
import functools

import jax
import jax.numpy as jnp
from jax import lax
from jax.experimental import pallas as pl
from jax.experimental.pallas import tpu as pltpu
from jax.experimental.pallas import tpu_sc as plsc

_N_USERS = 5000
_N_ITEMS = 5000
_N = _N_USERS + _N_ITEMS
_NPAD = 10240
_D = 128
_E = 320000
_NC, _NS, _L = 2, 16, 16
_NW = _NC * _NS
_EPW = _E // _NW
_BB = 125
_NB = _EPW // _BB
_RPW = _NPAD // _NW
_RPS = _NPAD // _NS
_CC = 64

_f32 = jnp.float32
_i32 = jnp.int32


def _wid():
    return lax.axis_index("s") * _NC + lax.axis_index("c")


_TCB = 1024


def _pre_tc_body(d0_ref, d1_ref, x_ref, dinv_ref, xs_ref):
    d = d0_ref[...] + d1_ref[...]
    dv = jnp.where(d > 0.5, lax.rsqrt(d), 0.0)
    dinv_ref[...] = dv
    xs_ref[...] = x_ref[...] * dv


def _comb_xs_tc_body(p0_ref, p1_ref, dv_ref, xs_ref, u_ref):
    dv = dv_ref[...]
    u = (p0_ref[...] + p1_ref[...]) * dv
    u_ref[...] = u
    xs_ref[...] = u * dv


def _su_first_tc_body(x0_ref, u_ref, so_ref):
    so_ref[...] = (x0_ref[...] + u_ref[...]) * 0.25


def _su_mid_tc_body(sin_ref, u_ref, so_ref):
    so_ref[...] = sin_ref[...] + u_ref[...] * 0.25


def _comb_last_tc_body(p0_ref, p1_ref, dv_ref, sin_ref, so_ref):
    dv = dv_ref[...]
    u = (p0_ref[...] + p1_ref[...]) * dv
    so_ref[...] = sin_ref[...] + u * 0.25


def _dv_spec():
    return pl.BlockSpec((_TCB, 1), lambda i: (i, 0))


def _mat_spec():
    return pl.BlockSpec((_TCB, _D), lambda i: (i, 0))


@functools.cache
def _build():
    mesh = plsc.VectorSubcoreMesh(core_axis_name="c", subcore_axis_name="s",
                                  num_cores=_NC)

    @functools.partial(
        pl.kernel,
        out_type=(
            jax.ShapeDtypeStruct((_NPAD,), _f32),
            jax.ShapeDtypeStruct((_NPAD,), _f32),
        ),
        mesh=mesh,
        scratch_types=[
            pltpu.VMEM((_NB, _BB), _i32),
            pltpu.VMEM((128,), _f32),
            pltpu.VMEM((_RPS,), _f32),
            pltpu.VMEM_SHARED((_NPAD,), _f32),
        ],
    )
    def deg_kernel(col_hbm, d0_hbm, d1_hbm, col_st, ones_v, z_v, deg_sh):
        cid = lax.axis_index("c")
        sid = lax.axis_index("s")
        w = _wid()
        pltpu.sync_copy(col_hbm.at[pl.ds(w * _NB, _NB)], col_st)
        for j in range(8):
            ones_v[pl.ds(j * _L, _L)] = jnp.ones((_L,), _f32)

        def zfill(i, c):
            z_v[pl.ds(i * _L, _L)] = jnp.zeros((_L,), _f32)
            return c

        lax.fori_loop(0, _RPS // _L, zfill, 0)
        pltpu.sync_copy(z_v, deg_sh.at[pl.ds(sid * _RPS, _RPS)])
        plsc.subcore_barrier()

        def body(b, c):
            pltpu.sync_copy(ones_v.at[pl.ds(0, _BB)],
                            deg_sh.at[col_st.at[b]], add=True)
            return c

        lax.fori_loop(0, _NB, body, 0)
        plsc.subcore_barrier()
        sl = pl.ds(sid * _RPS, _RPS)

        @pl.when(cid == 0)
        def _():
            pltpu.sync_copy(deg_sh.at[sl], d0_hbm.at[sl])

        @pl.when(cid == 1)
        def _():
            pltpu.sync_copy(deg_sh.at[sl], d1_hbm.at[sl])

    @functools.partial(
        pl.kernel,
        out_type=(
            jax.ShapeDtypeStruct((_NPAD, _D), _f32),
            jax.ShapeDtypeStruct((_NPAD, _D), _f32),
        ),
        mesh=mesh,
        scratch_types=[
            pltpu.VMEM((_NB, _BB), _i32),
            pltpu.VMEM((_BB, _D), _f32),
            pltpu.VMEM((_BB, _D), _f32),
            pltpu.VMEM((_BB,), _i32),
            pltpu.VMEM((_BB,), _i32),
            pltpu.VMEM((16, _D), _f32),
            pltpu.VMEM_SHARED((_NPAD, _D), _f32),
            pltpu.SemaphoreType.DMA,
            pltpu.SemaphoreType.DMA,
            pltpu.SemaphoreType.DMA,
            pltpu.SemaphoreType.DMA,
            pltpu.SemaphoreType.DMA,
            pltpu.SemaphoreType.DMA,
        ],
    )
    def layer_kernel(xs_hbm, row_hbm, col_hbm, p0_hbm, p1_hbm,
                     row_st, rbuf0, rbuf1, cbuf0, cbuf1, zbuf, acc_sh,
                     gsem0, gsem1, csem0, csem1, ssem0, ssem1):
        cid = lax.axis_index("c")
        sid = lax.axis_index("s")
        w = _wid()
        rbuf = (rbuf0, rbuf1)
        cbuf = (cbuf0, cbuf1)
        gsem = (gsem0, gsem1)
        csem = (csem0, csem1)
        ssem = (ssem0, ssem1)
        pltpu.async_copy(row_hbm.at[pl.ds(w * _NB, _NB)], row_st, gsem[1])

        def zfill(i, c):
            for j in range(_D // _L):
                zbuf[i, pl.ds(j * _L, _L)] = jnp.zeros((_L,), _f32)
            return c

        lax.fori_loop(0, 16, zfill, 0)

        def zcopy(c, cc):
            pltpu.async_copy(zbuf, acc_sh.at[pl.ds(sid * _RPS + c * 16, 16)],
                             ssem[0])
            return cc

        lax.fori_loop(0, _RPS // 16, zcopy, 0)
        pltpu.make_async_copy(row_hbm.at[pl.ds(w * _NB, _NB)], row_st,
                              gsem[1]).wait()
        pltpu.async_copy(col_hbm.at[w * _NB], cbuf[0], csem[0])

        def zdrain(c, cc):
            pltpu.make_async_copy(zbuf,
                                  acc_sh.at[pl.ds(sid * _RPS + c * 16, 16)],
                                  ssem[0]).wait()
            return cc

        lax.fori_loop(0, _RPS // 16, zdrain, 0)
        plsc.subcore_barrier()

        pltpu.async_copy(xs_hbm.at[row_st.at[0]], rbuf[0], gsem[0])

        def body(i, c):
            for p in range(2):
                b = i * 2 + p
                q = 1 - p

                @pl.when(b >= 1)
                def _():
                    pltpu.make_async_copy(rbuf[q], acc_sh.at[cbuf[q]],
                                          ssem[q]).wait()

                @pl.when(b + 1 < _NB)
                def _():
                    pltpu.async_copy(xs_hbm.at[row_st.at[b + 1]],
                                     rbuf[q], gsem[q])
                    pltpu.async_copy(col_hbm.at[w * _NB + b + 1],
                                     cbuf[q], csem[q])

                pltpu.make_async_copy(xs_hbm.at[row_st.at[b]],
                                      rbuf[p], gsem[p]).wait()
                pltpu.make_async_copy(col_hbm.at[w * _NB + b],
                                      cbuf[p], csem[p]).wait()
                pltpu.async_copy(rbuf[p], acc_sh.at[cbuf[p]], ssem[p],
                                 add=True)
            return c

        lax.fori_loop(0, _NB // 2, body, 0)
        pltpu.make_async_copy(rbuf[1], acc_sh.at[cbuf[1]], ssem[1]).wait()
        plsc.subcore_barrier()
        sl = pl.ds(sid * _RPS, _RPS)

        @pl.when(cid == 0)
        def _():
            pltpu.sync_copy(acc_sh.at[sl], p0_hbm.at[sl])

        @pl.when(cid == 1)
        def _():
            pltpu.sync_copy(acc_sh.at[sl], p1_hbm.at[sl])

    return deg_kernel, layer_kernel


def kernel(user_emb, item_emb, edge_index):
    deg_kernel, layer_kernel = _build()
    x0 = jnp.concatenate([user_emb, item_emb], axis=0)
    x0 = jnp.pad(x0, ((0, _NPAD - _N), (0, 0)))
    row = edge_index[0].astype(_i32).reshape(_E // _BB, _BB)
    col = edge_index[1].astype(_i32).reshape(_E // _BB, _BB)

    d0, d1 = deg_kernel(col)
    mat = jax.ShapeDtypeStruct((_NPAD, _D), _f32)
    dvt = jax.ShapeDtypeStruct((_NPAD, 1), _f32)
    grid = (_NPAD // _TCB,)
    dinv, xs = pl.pallas_call(
        _pre_tc_body,
        grid=grid,
        in_specs=[_dv_spec(), _dv_spec(), _mat_spec()],
        out_specs=[_dv_spec(), _mat_spec()],
        out_shape=(dvt, mat),
    )(d0.reshape(_NPAD, 1), d1.reshape(_NPAD, 1), x0)

    comb_xs = pl.pallas_call(
        _comb_xs_tc_body,
        grid=grid,
        in_specs=[_mat_spec(), _mat_spec(), _dv_spec()],
        out_specs=[_mat_spec(), _mat_spec()],
        out_shape=(mat, mat),
    )

    p0, p1 = layer_kernel(xs, row, col)
    xs, u = comb_xs(p0, p1, dinv)
    p0, p1 = layer_kernel(xs, row, col)
    ssum = pl.pallas_call(
        _su_first_tc_body,
        grid=grid,
        in_specs=[_mat_spec(), _mat_spec()],
        out_specs=_mat_spec(),
        out_shape=mat,
    )(x0, u)
    xs, u = comb_xs(p0, p1, dinv)
    p0, p1 = layer_kernel(xs, row, col)
    ssum = pl.pallas_call(
        _su_mid_tc_body,
        grid=grid,
        in_specs=[_mat_spec(), _mat_spec()],
        out_specs=_mat_spec(),
        out_shape=mat,
    )(ssum, u)
    ssum = pl.pallas_call(
        _comb_last_tc_body,
        grid=grid,
        in_specs=[_mat_spec(), _mat_spec(), _dv_spec(), _mat_spec()],
        out_specs=_mat_spec(),
        out_shape=mat,
    )(p0, p1, dinv, ssum)
    final = ssum[:_N]
    return final[:_N_USERS], final[_N_USERS:]


# speedup vs baseline: 1.0152x; 1.0152x over previous
import functools

import jax
import jax.numpy as jnp
from jax import lax
from jax.experimental import pallas as pl
from jax.experimental.pallas import tpu as pltpu
from jax.experimental.pallas import tpu_sc as plsc

_N_USERS = 5000
_N_ITEMS = 5000
_N = _N_USERS + _N_ITEMS
_NPAD = 10240
_D = 128
_E = 320000
_NC, _NS, _L = 2, 16, 16
_NW = _NC * _NS
_EPW = _E // _NW
_BB = 125
_NB = _EPW // _BB
_RPW = _NPAD // _NW
_RPS = _NPAD // _NS
_CC = 64

_f32 = jnp.float32
_i32 = jnp.int32


def _wid():
    return lax.axis_index("s") * _NC + lax.axis_index("c")


_TCB = 1000


def _pre_tc_body(d0_ref, d1_ref, u_ref, it_ref, dinv_ref, xs_ref):
    d = d0_ref[...] + d1_ref[...]
    dv = jnp.where(d > 0.5, lax.rsqrt(d), 0.0)
    dinv_ref[...] = dv
    x = jnp.where(pl.program_id(0) < 5, u_ref[...], it_ref[...])
    xs_ref[...] = x * dv


def _comb_first_tc_body(p0_ref, p1_ref, dv_ref, u0_ref, it0_ref,
                        xs_ref, so_ref):
    dv = dv_ref[...]
    u = (p0_ref[...] + p1_ref[...]) * dv
    xs_ref[...] = u * dv
    x0 = jnp.where(pl.program_id(0) < 5, u0_ref[...], it0_ref[...])
    so_ref[...] = (x0 + u) * 0.25


def _comb_mid_tc_body(p0_ref, p1_ref, dv_ref, sin_ref, xs_ref, so_ref):
    dv = dv_ref[...]
    u = (p0_ref[...] + p1_ref[...]) * dv
    xs_ref[...] = u * dv
    so_ref[...] = sin_ref[...] + u * 0.25


def _comb_last_tc_body(p0_ref, p1_ref, dv_ref, sin_ref, so_ref):
    dv = dv_ref[...]
    u = (p0_ref[...] + p1_ref[...]) * dv
    so_ref[...] = sin_ref[...] + u * 0.25


def _dv_spec(off=0):
    return pl.BlockSpec((_TCB, 1), lambda i: (i + off, 0))


def _mat_spec(off=0):
    return pl.BlockSpec((_TCB, _D), lambda i: (i + off, 0))


def _half_spec():
    return [pl.BlockSpec((_TCB, _D), lambda i: (jnp.minimum(i, 4), 0)),
            pl.BlockSpec((_TCB, _D), lambda i: (jnp.maximum(i - 5, 0), 0))]


@functools.cache
def _build():
    mesh = plsc.VectorSubcoreMesh(core_axis_name="c", subcore_axis_name="s",
                                  num_cores=_NC)

    @functools.partial(
        pl.kernel,
        out_type=(
            jax.ShapeDtypeStruct((_NPAD,), _f32),
            jax.ShapeDtypeStruct((_NPAD,), _f32),
        ),
        mesh=mesh,
        scratch_types=[
            pltpu.VMEM((_NB, _BB), _i32),
            pltpu.VMEM((128,), _f32),
            pltpu.VMEM((_RPS,), _f32),
            pltpu.VMEM_SHARED((_NPAD,), _f32),
        ],
    )
    def deg_kernel(col_hbm, d0_hbm, d1_hbm, col_st, ones_v, z_v, deg_sh):
        cid = lax.axis_index("c")
        sid = lax.axis_index("s")
        w = _wid()
        pltpu.sync_copy(col_hbm.at[pl.ds(w * _NB, _NB)], col_st)
        for j in range(8):
            ones_v[pl.ds(j * _L, _L)] = jnp.ones((_L,), _f32)

        def zfill(i, c):
            z_v[pl.ds(i * _L, _L)] = jnp.zeros((_L,), _f32)
            return c

        lax.fori_loop(0, _RPS // _L, zfill, 0)
        pltpu.sync_copy(z_v, deg_sh.at[pl.ds(sid * _RPS, _RPS)])
        plsc.subcore_barrier()

        def body(b, c):
            pltpu.sync_copy(ones_v.at[pl.ds(0, _BB)],
                            deg_sh.at[col_st.at[b]], add=True)
            return c

        lax.fori_loop(0, _NB, body, 0)
        plsc.subcore_barrier()
        sl = pl.ds(sid * _RPS, _RPS)

        @pl.when(cid == 0)
        def _():
            pltpu.sync_copy(deg_sh.at[sl], d0_hbm.at[sl])

        @pl.when(cid == 1)
        def _():
            pltpu.sync_copy(deg_sh.at[sl], d1_hbm.at[sl])

    @functools.partial(
        pl.kernel,
        out_type=(
            jax.ShapeDtypeStruct((_NPAD, _D), _f32),
            jax.ShapeDtypeStruct((_NPAD, _D), _f32),
        ),
        mesh=mesh,
        scratch_types=[
            pltpu.VMEM((_NB, _BB), _i32),
            pltpu.VMEM((_BB, _D), _f32),
            pltpu.VMEM((_BB, _D), _f32),
            pltpu.VMEM((_BB,), _i32),
            pltpu.VMEM((_BB,), _i32),
            pltpu.VMEM((16, _D), _f32),
            pltpu.VMEM_SHARED((_NPAD, _D), _f32),
            pltpu.SemaphoreType.DMA,
            pltpu.SemaphoreType.DMA,
            pltpu.SemaphoreType.DMA,
            pltpu.SemaphoreType.DMA,
            pltpu.SemaphoreType.DMA,
            pltpu.SemaphoreType.DMA,
        ],
    )
    def layer_kernel(xs_hbm, row_hbm, col_hbm, p0_hbm, p1_hbm,
                     row_st, rbuf0, rbuf1, cbuf0, cbuf1, zbuf, acc_sh,
                     gsem0, gsem1, csem0, csem1, ssem0, ssem1):
        cid = lax.axis_index("c")
        sid = lax.axis_index("s")
        w = _wid()
        rbuf = (rbuf0, rbuf1)
        cbuf = (cbuf0, cbuf1)
        gsem = (gsem0, gsem1)
        csem = (csem0, csem1)
        ssem = (ssem0, ssem1)
        pltpu.async_copy(row_hbm.at[pl.ds(w * _NB, _NB)], row_st, gsem[1])

        def zfill(i, c):
            for j in range(_D // _L):
                zbuf[i, pl.ds(j * _L, _L)] = jnp.zeros((_L,), _f32)
            return c

        lax.fori_loop(0, 16, zfill, 0)

        def zcopy(c, cc):
            pltpu.async_copy(zbuf, acc_sh.at[pl.ds(sid * _RPS + c * 16, 16)],
                             ssem[0])
            return cc

        lax.fori_loop(0, _RPS // 16, zcopy, 0)
        pltpu.make_async_copy(row_hbm.at[pl.ds(w * _NB, _NB)], row_st,
                              gsem[1]).wait()
        pltpu.async_copy(col_hbm.at[w * _NB], cbuf[0], csem[0])

        def zdrain(c, cc):
            pltpu.make_async_copy(zbuf,
                                  acc_sh.at[pl.ds(sid * _RPS + c * 16, 16)],
                                  ssem[0]).wait()
            return cc

        lax.fori_loop(0, _RPS // 16, zdrain, 0)
        plsc.subcore_barrier()

        pltpu.async_copy(xs_hbm.at[row_st.at[0]], rbuf[0], gsem[0])

        def body(i, c):
            for p in range(2):
                b = i * 2 + p
                q = 1 - p

                @pl.when(b >= 1)
                def _():
                    pltpu.make_async_copy(rbuf[q], acc_sh.at[cbuf[q]],
                                          ssem[q]).wait()

                @pl.when(b + 1 < _NB)
                def _():
                    pltpu.async_copy(xs_hbm.at[row_st.at[b + 1]],
                                     rbuf[q], gsem[q])
                    pltpu.async_copy(col_hbm.at[w * _NB + b + 1],
                                     cbuf[q], csem[q])

                pltpu.make_async_copy(xs_hbm.at[row_st.at[b]],
                                      rbuf[p], gsem[p]).wait()
                pltpu.make_async_copy(col_hbm.at[w * _NB + b],
                                      cbuf[p], csem[p]).wait()
                pltpu.async_copy(rbuf[p], acc_sh.at[cbuf[p]], ssem[p],
                                 add=True)
            return c

        lax.fori_loop(0, _NB // 2, body, 0)
        pltpu.make_async_copy(rbuf[1], acc_sh.at[cbuf[1]], ssem[1]).wait()
        plsc.subcore_barrier()
        sl = pl.ds(sid * _RPS, _RPS)

        @pl.when(cid == 0)
        def _():
            pltpu.sync_copy(acc_sh.at[sl], p0_hbm.at[sl])

        @pl.when(cid == 1)
        def _():
            pltpu.sync_copy(acc_sh.at[sl], p1_hbm.at[sl])

    return deg_kernel, layer_kernel


def kernel(user_emb, item_emb, edge_index):
    deg_kernel, layer_kernel = _build()
    row = edge_index[0].astype(_i32).reshape(_E // _BB, _BB)
    col = edge_index[1].astype(_i32).reshape(_E // _BB, _BB)

    d0, d1 = deg_kernel(col)
    mat = jax.ShapeDtypeStruct((_NPAD, _D), _f32)
    half = jax.ShapeDtypeStruct((_N_USERS, _D), _f32)
    dvt = jax.ShapeDtypeStruct((_NPAD, 1), _f32)
    grid = (_N // _TCB,)
    dinv, xs = pl.pallas_call(
        _pre_tc_body,
        grid=grid,
        in_specs=[_dv_spec(), _dv_spec()] + _half_spec(),
        out_specs=[_dv_spec(), _mat_spec()],
        out_shape=(dvt, mat),
    )(d0.reshape(_NPAD, 1), d1.reshape(_NPAD, 1), user_emb, item_emb)

    p0, p1 = layer_kernel(xs, row, col)
    xs, ssum = pl.pallas_call(
        _comb_first_tc_body,
        grid=grid,
        in_specs=[_mat_spec(), _mat_spec(), _dv_spec()] + _half_spec(),
        out_specs=[_mat_spec(), _mat_spec()],
        out_shape=(mat, mat),
    )(p0, p1, dinv, user_emb, item_emb)

    p0, p1 = layer_kernel(xs, row, col)
    xs, ssum = pl.pallas_call(
        _comb_mid_tc_body,
        grid=grid,
        in_specs=[_mat_spec(), _mat_spec(), _dv_spec(), _mat_spec()],
        out_specs=[_mat_spec(), _mat_spec()],
        out_shape=(mat, mat),
    )(p0, p1, dinv, ssum)

    p0, p1 = layer_kernel(xs, row, col)
    outs = []
    for off in (0, 5):
        outs.append(pl.pallas_call(
            _comb_last_tc_body,
            grid=(_N_USERS // _TCB,),
            in_specs=[_mat_spec(off), _mat_spec(off), _dv_spec(off),
                      _mat_spec(off)],
            out_specs=_mat_spec(),
            out_shape=half,
        )(p0, p1, dinv, ssum))
    return outs[0], outs[1]
